# Initial kernel scaffold; baseline (speedup 1.0000x reference)
import jax
import jax.numpy as jnp
from jax.experimental import pallas as pl


def kernel(in_feat, edge_index_r0, edge_index_r1, W1, b1, W2, b2, W3, b3, W4, b4):
    raise NotImplementedError("write your pallas kernel here")


# speedup vs baseline: 3.4369x; 3.4369x over previous
import functools

import jax
import jax.numpy as jnp
from jax import lax
from jax.experimental import pallas as pl
from jax.experimental.pallas import tpu as pltpu
from jax.experimental.pallas import tpu_sc as plsc

_N = 10000
_E = 320000
_H = 64
_NPAD = 10016
_EPAD = 327680
_NTILES = 32
_CPT = 80
_BLK = 1000


def _mesh():
    return plsc.VectorSubcoreMesh(core_axis_name="c", subcore_axis_name="s")


_SC_PARAMS = pltpu.CompilerParams(use_tc_tiling_on_sc=False)


def _sc_segsum(y, src3, dst3):
    @functools.partial(
        pl.kernel,
        out_type=jax.ShapeDtypeStruct((2, _N, _H), jnp.float32),
        mesh=_mesh(),
        scratch_types=[
            pltpu.VMEM((8, 128), jnp.int32),
            pltpu.VMEM((8, 128), jnp.int32),
            pltpu.VMEM((128, _H), jnp.float32),
            pltpu.VMEM_SHARED((_NPAD, _H), jnp.float32),
        ],
        compiler_params=_SC_PARAMS,
    )
    def k(y_hbm, src_hbm, dst_hbm, out_hbm, sidx, didx, rows, acc):
        cid = lax.axis_index("c")
        sid = lax.axis_index("s")
        w = sid * 2 + cid
        zero = jnp.zeros((16,), jnp.float32)

        @pl.loop(0, 128)
        def _(r):
            for c4 in range(4):
                rows[r, pl.ds(c4 * 16, 16)] = zero

        base = sid * 624
        for t in range(4):
            pltpu.sync_copy(rows, acc.at[pl.ds(base + t * 128, 128)])
        pltpu.sync_copy(rows.at[pl.ds(0, 112)], acc.at[pl.ds(base + 512, 112)])

        @pl.when(sid == 15)
        def _():
            pltpu.sync_copy(rows.at[pl.ds(0, 32)], acc.at[pl.ds(9984, 32)])

        plsc.subcore_barrier()

        @pl.loop(0, _CPT // 8)
        def _(g):
            pltpu.sync_copy(src_hbm.at[w, pl.ds(g * 8, 8)], sidx)
            pltpu.sync_copy(dst_hbm.at[w, pl.ds(g * 8, 8)], didx)
            for j in range(8):
                pltpu.sync_copy(y_hbm.at[sidx.at[j]], rows)
                pltpu.sync_copy(rows, acc.at[didx.at[j]], add=True)

        plsc.subcore_barrier()
        ob = sid * 624
        pltpu.sync_copy(acc.at[pl.ds(ob, 624)], out_hbm.at[cid, pl.ds(ob, 624)])

        @pl.when(sid == 15)
        def _():
            pltpu.sync_copy(acc.at[pl.ds(9984, 16)], out_hbm.at[cid, pl.ds(9984, 16)])

    return k(y, src3, dst3)


def _sc_deg(dst_both):
    @functools.partial(
        pl.kernel,
        out_type=jax.ShapeDtypeStruct((2, _N, 16), jnp.float32),
        mesh=_mesh(),
        scratch_types=[
            pltpu.VMEM((8, 128), jnp.int32),
            pltpu.VMEM((128, 16), jnp.float32),
            pltpu.VMEM_SHARED((_NPAD, 16), jnp.float32),
        ],
        compiler_params=_SC_PARAMS,
    )
    def k(dst_hbm, out_hbm, didx, rows, acc):
        cid = lax.axis_index("c")
        sid = lax.axis_index("s")
        zero = jnp.zeros((16,), jnp.float32)
        one = jnp.ones((16,), jnp.float32)

        @pl.loop(0, 128)
        def _(r):
            rows[r, pl.ds(0, 16)] = zero

        base = sid * 624
        for t in range(4):
            pltpu.sync_copy(rows, acc.at[pl.ds(base + t * 128, 128)])
        pltpu.sync_copy(rows.at[pl.ds(0, 112)], acc.at[pl.ds(base + 512, 112)])

        @pl.when(sid == 15)
        def _():
            pltpu.sync_copy(rows.at[pl.ds(0, 32)], acc.at[pl.ds(9984, 32)])

        @pl.loop(0, 128)
        def _(r):
            rows[r, pl.ds(0, 16)] = one

        plsc.subcore_barrier()

        @pl.loop(0, 20)
        def _(g):
            pltpu.sync_copy(dst_hbm.at[cid, sid, pl.ds(g * 8, 8)], didx)
            for j in range(8):
                pltpu.sync_copy(rows, acc.at[didx.at[j]], add=True)

        plsc.subcore_barrier()
        ob = sid * 624
        pltpu.sync_copy(acc.at[pl.ds(ob, 624)], out_hbm.at[cid, pl.ds(ob, 624)])

        @pl.when(sid == 15)
        def _():
            pltpu.sync_copy(acc.at[pl.ds(9984, 16)], out_hbm.at[cid, pl.ds(9984, 16)])

    return k(dst_both)


def _lrelu(x):
    return jnp.where(x >= 0, x, 0.01 * x)


def _dinv(deg_ref):
    return lax.rsqrt(jnp.maximum(deg_ref[:, 0:1], 1.0))


def _mlp(x, W1, b1, W2, b2):
    def body(x_ref, w1_ref, b1_ref, w2_ref, b2_ref, o_ref):
        h = jnp.dot(x_ref[...], w1_ref[...], preferred_element_type=jnp.float32, precision=lax.Precision.HIGHEST)
        h = _lrelu(h + b1_ref[...])
        h = jnp.dot(h, w2_ref[...], preferred_element_type=jnp.float32, precision=lax.Precision.HIGHEST)
        o_ref[...] = _lrelu(h + b2_ref[...])

    return pl.pallas_call(
        body,
        grid=(_N // _BLK,),
        in_specs=[
            pl.BlockSpec((_BLK, 128), lambda i: (i, 0)),
            pl.BlockSpec((128, _H), lambda i: (0, 0)),
            pl.BlockSpec((1, _H), lambda i: (0, 0)),
            pl.BlockSpec((_H, _H), lambda i: (0, 0)),
            pl.BlockSpec((1, _H), lambda i: (0, 0)),
        ],
        out_specs=pl.BlockSpec((_BLK, _H), lambda i: (i, 0)),
        out_shape=jax.ShapeDtypeStruct((_N, _H), jnp.float32),
    )(x, W1, b1.reshape(1, _H), W2, b2.reshape(1, _H))


def _prep(h, deg):
    def body(h_ref, deg_ref, y_ref):
        y_ref[...] = h_ref[...] * _dinv(deg_ref)

    return pl.pallas_call(
        body,
        grid=(_N // _BLK,),
        in_specs=[
            pl.BlockSpec((_BLK, _H), lambda i: (i, 0)),
            pl.BlockSpec((_BLK, 16), lambda i: (i, 0)),
        ],
        out_specs=pl.BlockSpec((_BLK, _H), lambda i: (i, 0)),
        out_shape=jax.ShapeDtypeStruct((_N, _H), jnp.float32),
    )(h, deg)


def _mid(f, p, deg):
    def body(f_ref, p_ref, deg_ref, f1_ref, y1_ref):
        di = _dinv(deg_ref)
        f1 = f_ref[...] - (p_ref[0] + p_ref[1]) * di
        f1_ref[...] = f1
        y1_ref[...] = f1 * di

    return pl.pallas_call(
        body,
        grid=(_N // _BLK,),
        in_specs=[
            pl.BlockSpec((_BLK, _H), lambda i: (i, 0)),
            pl.BlockSpec((2, _BLK, _H), lambda i: (0, i, 0)),
            pl.BlockSpec((_BLK, 16), lambda i: (i, 0)),
        ],
        out_specs=[
            pl.BlockSpec((_BLK, _H), lambda i: (i, 0)),
            pl.BlockSpec((_BLK, _H), lambda i: (i, 0)),
        ],
        out_shape=[
            jax.ShapeDtypeStruct((_N, _H), jnp.float32),
            jax.ShapeDtypeStruct((_N, _H), jnp.float32),
        ],
    )(f, p, deg)


def _end(h, f1, p, deg, W3s, b3):
    def body(h_ref, f1_ref, p_ref, deg_ref, w3_ref, b3_ref, o_ref):
        f2 = f1_ref[...] - (p_ref[0] + p_ref[1]) * _dinv(deg_ref)
        o = jnp.dot(h_ref[...], w3_ref[0], preferred_element_type=jnp.float32, precision=lax.Precision.HIGHEST)
        o += jnp.dot(f1_ref[...], w3_ref[1], preferred_element_type=jnp.float32, precision=lax.Precision.HIGHEST)
        o += jnp.dot(f2, w3_ref[2], preferred_element_type=jnp.float32, precision=lax.Precision.HIGHEST)
        o_ref[...] = o + b3_ref[...]

    return pl.pallas_call(
        body,
        grid=(_N // _BLK,),
        in_specs=[
            pl.BlockSpec((_BLK, _H), lambda i: (i, 0)),
            pl.BlockSpec((_BLK, _H), lambda i: (i, 0)),
            pl.BlockSpec((2, _BLK, _H), lambda i: (0, i, 0)),
            pl.BlockSpec((_BLK, 16), lambda i: (i, 0)),
            pl.BlockSpec((3, _H, _H), lambda i: (0, 0, 0)),
            pl.BlockSpec((1, _H), lambda i: (0, 0)),
        ],
        out_specs=pl.BlockSpec((_BLK, _H), lambda i: (i, 0)),
        out_shape=jax.ShapeDtypeStruct((_N, _H), jnp.float32),
    )(h, f1, p, deg, W3s, b3)


def _final(ha, hb, W4p, b4p):
    def body(a_ref, b_ref, w_ref, bias_ref, o_ref):
        s = _lrelu(a_ref[...] + b_ref[...])
        o_ref[...] = (
            jnp.dot(s, w_ref[...], preferred_element_type=jnp.float32, precision=lax.Precision.HIGHEST)
            + bias_ref[...]
        )

    return pl.pallas_call(
        body,
        grid=(_N // _BLK,),
        in_specs=[
            pl.BlockSpec((_BLK, _H), lambda i: (i, 0)),
            pl.BlockSpec((_BLK, _H), lambda i: (i, 0)),
            pl.BlockSpec((_H, 128), lambda i: (0, 0)),
            pl.BlockSpec((1, 128), lambda i: (0, 0)),
        ],
        out_specs=pl.BlockSpec((_BLK, 128), lambda i: (i, 0)),
        out_shape=jax.ShapeDtypeStruct((_N, 128), jnp.float32),
    )(ha, hb, W4p, b4p)


def kernel(in_feat, edge_index_r0, edge_index_r1, W1, b1, W2, b2, W3, b3, W4, b4):
    pad = _EPAD - _E

    def prep_edges(ei):
        src = jnp.concatenate([ei[0], jnp.zeros((pad,), jnp.int32)])
        dst = jnp.concatenate([ei[1], jnp.full((pad,), _N, jnp.int32)])
        return (
            src.reshape(_NTILES, _CPT, 128),
            dst.reshape(_NTILES, _CPT, 128),
            dst.reshape(16, 160, 128),
        )

    src0, dst0, dstd0 = prep_edges(edge_index_r0)
    src1, dst1, dstd1 = prep_edges(edge_index_r1)

    deg_both = _sc_deg(jnp.stack([dstd0, dstd1]))
    h = _mlp(in_feat, W1, b1, W2, b2)

    W3s = jnp.stack([
        3.0 * W3[:_H],
        -3.0 * W3[:_H] + 3.0 * W3[_H:2 * _H],
        0.75 * W3[:_H] - 1.5 * W3[_H:2 * _H] + 0.75 * W3[2 * _H:],
    ])
    b3r = b3.reshape(1, _H)

    h_all = []
    for r, (src3, dst3) in enumerate(((src0, dst0), (src1, dst1))):
        deg = deg_both[r]
        y0 = _prep(h, deg)
        p1 = _sc_segsum(y0, src3, dst3)
        f1, y1 = _mid(h, p1, deg)
        p2 = _sc_segsum(y1, src3, dst3)
        h = _end(h, f1, p2, deg, W3s, b3r)
        h_all.append(h)

    W4p = jnp.zeros((_H, 128), jnp.float32).at[:, :2].set(W4)
    b4p = jnp.zeros((1, 128), jnp.float32).at[0, :2].set(b4)
    out = _final(h_all[0], h_all[1], W4p, b4p)
    return out[:, :2]


# speedup vs baseline: 4.0220x; 1.1702x over previous
import functools

import jax
import jax.numpy as jnp
from jax import lax
from jax.experimental import pallas as pl
from jax.experimental.pallas import tpu as pltpu
from jax.experimental.pallas import tpu_sc as plsc

_N = 10000
_E = 320000
_H = 64
_NPAD = 10016
_EPAD = 327680
_NTILES = 32
_CPT = 80
_BLK = 1000


def _mesh():
    return plsc.VectorSubcoreMesh(core_axis_name="c", subcore_axis_name="s")


_SC_PARAMS = pltpu.CompilerParams(use_tc_tiling_on_sc=False)


def _sc_segsum(y, src3, dst3):
    @functools.partial(
        pl.kernel,
        out_type=jax.ShapeDtypeStruct((2, _N, _H), jnp.float32),
        mesh=_mesh(),
        scratch_types=[
            pltpu.VMEM((_CPT, 128), jnp.int32),
            pltpu.VMEM((_CPT, 128), jnp.int32),
            pltpu.VMEM((128, _H), jnp.float32),
            pltpu.VMEM((128, _H), jnp.float32),
            pltpu.VMEM((128, _H), jnp.float32),
            pltpu.VMEM((128, _H), jnp.float32),
            pltpu.VMEM_SHARED((_NPAD, _H), jnp.float32),
            pltpu.SemaphoreType.DMA,
            pltpu.SemaphoreType.DMA,
            pltpu.SemaphoreType.DMA,
            pltpu.SemaphoreType.DMA,
            pltpu.SemaphoreType.DMA,
            pltpu.SemaphoreType.DMA,
            pltpu.SemaphoreType.DMA,
            pltpu.SemaphoreType.DMA,
        ],
        compiler_params=_SC_PARAMS,
    )
    def k(y_hbm, src_hbm, dst_hbm, out_hbm, sidx, didx,
          r0, r1, r2, r3, acc, g0, g1, g2, g3, s0, s1, s2, s3):
        cid = lax.axis_index("c")
        sid = lax.axis_index("s")
        w = sid * 2 + cid
        rows = (r0, r1, r2, r3)
        gsem = (g0, g1, g2, g3)
        ssem = (s0, s1, s2, s3)
        zero = jnp.zeros((16,), jnp.float32)

        @pl.loop(0, 128)
        def _(r):
            for c4 in range(4):
                r0[r, pl.ds(c4 * 16, 16)] = zero

        base = sid * 624
        for t in range(4):
            pltpu.sync_copy(r0, acc.at[pl.ds(base + t * 128, 128)])
        pltpu.sync_copy(r0.at[pl.ds(0, 112)], acc.at[pl.ds(base + 512, 112)])

        @pl.when(sid == 15)
        def _():
            pltpu.sync_copy(r0.at[pl.ds(0, 32)], acc.at[pl.ds(9984, 32)])

        pltpu.sync_copy(src_hbm.at[w], sidx)
        pltpu.sync_copy(dst_hbm.at[w], didx)
        for j in range(4):
            pltpu.async_copy(y_hbm.at[sidx.at[j]], rows[j], gsem[j])

        plsc.subcore_barrier()

        @pl.loop(0, _CPT // 4 - 1)
        def _(g):
            for j in range(4):
                c = g * 4 + j
                pltpu.make_async_copy(y_hbm.at[sidx.at[0]], rows[j], gsem[j]).wait()
                pltpu.async_copy(rows[j], acc.at[didx.at[c]], ssem[j], add=True)
            for j in range(4):
                c4 = g * 4 + 4 + j
                pltpu.make_async_copy(rows[j], acc.at[didx.at[0]], ssem[j]).wait()
                pltpu.async_copy(y_hbm.at[sidx.at[c4]], rows[j], gsem[j])

        for j in range(4):
            c = _CPT - 4 + j
            pltpu.make_async_copy(y_hbm.at[sidx.at[0]], rows[j], gsem[j]).wait()
            pltpu.async_copy(rows[j], acc.at[didx.at[c]], ssem[j], add=True)
        for j in range(4):
            pltpu.make_async_copy(rows[j], acc.at[didx.at[0]], ssem[j]).wait()

        plsc.subcore_barrier()
        ob = sid * 624
        pltpu.sync_copy(acc.at[pl.ds(ob, 624)], out_hbm.at[cid, pl.ds(ob, 624)])

        @pl.when(sid == 15)
        def _():
            pltpu.sync_copy(acc.at[pl.ds(9984, 16)], out_hbm.at[cid, pl.ds(9984, 16)])

    return k(y, src3, dst3)


def _sc_deg(dst_both):
    @functools.partial(
        pl.kernel,
        out_type=jax.ShapeDtypeStruct((2, _N, 16), jnp.float32),
        mesh=_mesh(),
        scratch_types=[
            pltpu.VMEM((8, 128), jnp.int32),
            pltpu.VMEM((128, 16), jnp.float32),
            pltpu.VMEM_SHARED((_NPAD, 16), jnp.float32),
        ],
        compiler_params=_SC_PARAMS,
    )
    def k(dst_hbm, out_hbm, didx, rows, acc):
        cid = lax.axis_index("c")
        sid = lax.axis_index("s")
        zero = jnp.zeros((16,), jnp.float32)
        one = jnp.ones((16,), jnp.float32)

        @pl.loop(0, 128)
        def _(r):
            rows[r, pl.ds(0, 16)] = zero

        base = sid * 624
        for t in range(4):
            pltpu.sync_copy(rows, acc.at[pl.ds(base + t * 128, 128)])
        pltpu.sync_copy(rows.at[pl.ds(0, 112)], acc.at[pl.ds(base + 512, 112)])

        @pl.when(sid == 15)
        def _():
            pltpu.sync_copy(rows.at[pl.ds(0, 32)], acc.at[pl.ds(9984, 32)])

        @pl.loop(0, 128)
        def _(r):
            rows[r, pl.ds(0, 16)] = one

        plsc.subcore_barrier()

        @pl.loop(0, 20)
        def _(g):
            pltpu.sync_copy(dst_hbm.at[cid, sid, pl.ds(g * 8, 8)], didx)
            for j in range(8):
                pltpu.sync_copy(rows, acc.at[didx.at[j]], add=True)

        plsc.subcore_barrier()
        ob = sid * 624
        pltpu.sync_copy(acc.at[pl.ds(ob, 624)], out_hbm.at[cid, pl.ds(ob, 624)])

        @pl.when(sid == 15)
        def _():
            pltpu.sync_copy(acc.at[pl.ds(9984, 16)], out_hbm.at[cid, pl.ds(9984, 16)])

    return k(dst_both)


def _lrelu(x):
    return jnp.where(x >= 0, x, 0.01 * x)


def _dinv(deg_ref):
    return lax.rsqrt(jnp.maximum(deg_ref[:, 0:1], 1.0))


def _mlp(x, W1, b1, W2, b2):
    def body(x_ref, w1_ref, b1_ref, w2_ref, b2_ref, o_ref):
        h = jnp.dot(x_ref[...], w1_ref[...], preferred_element_type=jnp.float32, precision=lax.Precision.HIGHEST)
        h = _lrelu(h + b1_ref[...])
        h = jnp.dot(h, w2_ref[...], preferred_element_type=jnp.float32, precision=lax.Precision.HIGHEST)
        o_ref[...] = _lrelu(h + b2_ref[...])

    return pl.pallas_call(
        body,
        grid=(_N // _BLK,),
        in_specs=[
            pl.BlockSpec((_BLK, 128), lambda i: (i, 0)),
            pl.BlockSpec((128, _H), lambda i: (0, 0)),
            pl.BlockSpec((1, _H), lambda i: (0, 0)),
            pl.BlockSpec((_H, _H), lambda i: (0, 0)),
            pl.BlockSpec((1, _H), lambda i: (0, 0)),
        ],
        out_specs=pl.BlockSpec((_BLK, _H), lambda i: (i, 0)),
        out_shape=jax.ShapeDtypeStruct((_N, _H), jnp.float32),
    )(x, W1, b1.reshape(1, _H), W2, b2.reshape(1, _H))


def _prep(h, deg):
    def body(h_ref, deg_ref, y_ref):
        y_ref[...] = h_ref[...] * _dinv(deg_ref)

    return pl.pallas_call(
        body,
        grid=(_N // _BLK,),
        in_specs=[
            pl.BlockSpec((_BLK, _H), lambda i: (i, 0)),
            pl.BlockSpec((_BLK, 16), lambda i: (i, 0)),
        ],
        out_specs=pl.BlockSpec((_BLK, _H), lambda i: (i, 0)),
        out_shape=jax.ShapeDtypeStruct((_N, _H), jnp.float32),
    )(h, deg)


def _mid(f, p, deg):
    def body(f_ref, p_ref, deg_ref, f1_ref, y1_ref):
        di = _dinv(deg_ref)
        f1 = f_ref[...] - (p_ref[0] + p_ref[1]) * di
        f1_ref[...] = f1
        y1_ref[...] = f1 * di

    return pl.pallas_call(
        body,
        grid=(_N // _BLK,),
        in_specs=[
            pl.BlockSpec((_BLK, _H), lambda i: (i, 0)),
            pl.BlockSpec((2, _BLK, _H), lambda i: (0, i, 0)),
            pl.BlockSpec((_BLK, 16), lambda i: (i, 0)),
        ],
        out_specs=[
            pl.BlockSpec((_BLK, _H), lambda i: (i, 0)),
            pl.BlockSpec((_BLK, _H), lambda i: (i, 0)),
        ],
        out_shape=[
            jax.ShapeDtypeStruct((_N, _H), jnp.float32),
            jax.ShapeDtypeStruct((_N, _H), jnp.float32),
        ],
    )(f, p, deg)


def _end(h, f1, p, deg, W3s, b3):
    def body(h_ref, f1_ref, p_ref, deg_ref, w3_ref, b3_ref, o_ref):
        f2 = f1_ref[...] - (p_ref[0] + p_ref[1]) * _dinv(deg_ref)
        o = jnp.dot(h_ref[...], w3_ref[0], preferred_element_type=jnp.float32, precision=lax.Precision.HIGHEST)
        o += jnp.dot(f1_ref[...], w3_ref[1], preferred_element_type=jnp.float32, precision=lax.Precision.HIGHEST)
        o += jnp.dot(f2, w3_ref[2], preferred_element_type=jnp.float32, precision=lax.Precision.HIGHEST)
        o_ref[...] = o + b3_ref[...]

    return pl.pallas_call(
        body,
        grid=(_N // _BLK,),
        in_specs=[
            pl.BlockSpec((_BLK, _H), lambda i: (i, 0)),
            pl.BlockSpec((_BLK, _H), lambda i: (i, 0)),
            pl.BlockSpec((2, _BLK, _H), lambda i: (0, i, 0)),
            pl.BlockSpec((_BLK, 16), lambda i: (i, 0)),
            pl.BlockSpec((3, _H, _H), lambda i: (0, 0, 0)),
            pl.BlockSpec((1, _H), lambda i: (0, 0)),
        ],
        out_specs=pl.BlockSpec((_BLK, _H), lambda i: (i, 0)),
        out_shape=jax.ShapeDtypeStruct((_N, _H), jnp.float32),
    )(h, f1, p, deg, W3s, b3)


def _final(ha, hb, W4p, b4p):
    def body(a_ref, b_ref, w_ref, bias_ref, o_ref):
        s = _lrelu(a_ref[...] + b_ref[...])
        o_ref[...] = (
            jnp.dot(s, w_ref[...], preferred_element_type=jnp.float32, precision=lax.Precision.HIGHEST)
            + bias_ref[...]
        )

    return pl.pallas_call(
        body,
        grid=(_N // _BLK,),
        in_specs=[
            pl.BlockSpec((_BLK, _H), lambda i: (i, 0)),
            pl.BlockSpec((_BLK, _H), lambda i: (i, 0)),
            pl.BlockSpec((_H, 128), lambda i: (0, 0)),
            pl.BlockSpec((1, 128), lambda i: (0, 0)),
        ],
        out_specs=pl.BlockSpec((_BLK, 128), lambda i: (i, 0)),
        out_shape=jax.ShapeDtypeStruct((_N, 128), jnp.float32),
    )(ha, hb, W4p, b4p)


def kernel(in_feat, edge_index_r0, edge_index_r1, W1, b1, W2, b2, W3, b3, W4, b4):
    pad = _EPAD - _E

    def prep_edges(ei):
        src = jnp.concatenate([ei[0], jnp.zeros((pad,), jnp.int32)])
        dst = jnp.concatenate([ei[1], jnp.full((pad,), _N, jnp.int32)])
        return (
            src.reshape(_NTILES, _CPT, 128),
            dst.reshape(_NTILES, _CPT, 128),
            dst.reshape(16, 160, 128),
        )

    src0, dst0, dstd0 = prep_edges(edge_index_r0)
    src1, dst1, dstd1 = prep_edges(edge_index_r1)

    deg_both = _sc_deg(jnp.stack([dstd0, dstd1]))
    h = _mlp(in_feat, W1, b1, W2, b2)

    W3s = jnp.stack([
        3.0 * W3[:_H],
        -3.0 * W3[:_H] + 3.0 * W3[_H:2 * _H],
        0.75 * W3[:_H] - 1.5 * W3[_H:2 * _H] + 0.75 * W3[2 * _H:],
    ])
    b3r = b3.reshape(1, _H)

    h_all = []
    for r, (src3, dst3) in enumerate(((src0, dst0), (src1, dst1))):
        deg = deg_both[r]
        y0 = _prep(h, deg)
        p1 = _sc_segsum(y0, src3, dst3)
        f1, y1 = _mid(h, p1, deg)
        p2 = _sc_segsum(y1, src3, dst3)
        h = _end(h, f1, p2, deg, W3s, b3r)
        h_all.append(h)

    W4p = jnp.zeros((_H, 128), jnp.float32).at[:, :2].set(W4)
    b4p = jnp.zeros((1, 128), jnp.float32).at[0, :2].set(b4)
    out = _final(h_all[0], h_all[1], W4p, b4p)
    return out[:, :2]


# speedup vs baseline: 8.7515x; 2.1759x over previous
import functools

import jax
import jax.numpy as jnp
from jax import lax
from jax.experimental import pallas as pl
from jax.experimental.pallas import tpu as pltpu
from jax.experimental.pallas import tpu_sc as plsc

_N = 10000
_E = 320000
_H = 64
_NPAD = 10016
_EPAD = 327680
_NTILES = 32
_CPC = 160
_BLK = 1000


def _mesh():
    return plsc.VectorSubcoreMesh(core_axis_name="c", subcore_axis_name="s")


_SC_PARAMS = pltpu.CompilerParams(use_tc_tiling_on_sc=False)


def _sc_segsum(y2, src3, dst3):
    @functools.partial(
        pl.kernel,
        out_type=jax.ShapeDtypeStruct((2, _N, 32), jnp.float32),
        mesh=_mesh(),
        scratch_types=[
            pltpu.VMEM((160, 128), jnp.int32),
            pltpu.VMEM((160, 128), jnp.int32),
            pltpu.VMEM((128, 32), jnp.float32),
            pltpu.VMEM((128, 32), jnp.float32),
            pltpu.VMEM((128, 32), jnp.float32),
            pltpu.VMEM((128, 32), jnp.float32),
            pltpu.VMEM_SHARED((_NPAD, 32), jnp.float32),
            pltpu.VMEM_SHARED((_N, 32), jnp.float32),
            pltpu.SemaphoreType.DMA,
            pltpu.SemaphoreType.DMA,
            pltpu.SemaphoreType.DMA,
            pltpu.SemaphoreType.DMA,
            pltpu.SemaphoreType.DMA,
            pltpu.SemaphoreType.DMA,
            pltpu.SemaphoreType.DMA,
            pltpu.SemaphoreType.DMA,
        ],
        compiler_params=_SC_PARAMS,
    )
    def k(y_hbm, src_hbm, dst_hbm, out_hbm, sidx, didx,
          r0, r1, r2, r3, acc, ycopy, g0, g1, g2, g3, s0, s1, s2, s3):
        cid = lax.axis_index("c")
        sid = lax.axis_index("s")
        rows = (r0, r1, r2, r3)
        gsem = (g0, g1, g2, g3)
        ssem = (s0, s1, s2, s3)
        zero = jnp.zeros((16,), jnp.float32)

        @pl.loop(0, 128)
        def _(r):
            for c2 in range(2):
                r0[r, pl.ds(c2 * 16, 16)] = zero

        base = sid * 624
        for t in range(4):
            pltpu.sync_copy(r0, acc.at[pl.ds(base + t * 128, 128)])
        pltpu.sync_copy(r0.at[pl.ds(0, 112)], acc.at[pl.ds(base + 512, 112)])

        @pl.when(sid == 15)
        def _():
            pltpu.sync_copy(r0.at[pl.ds(0, 32)], acc.at[pl.ds(9984, 32)])

        pltpu.sync_copy(y_hbm.at[cid, pl.ds(base, 624)], ycopy.at[pl.ds(base, 624)])

        @pl.when(sid == 15)
        def _():
            pltpu.sync_copy(y_hbm.at[cid, pl.ds(9984, 16)], ycopy.at[pl.ds(9984, 16)])

        pltpu.sync_copy(src_hbm.at[sid], sidx)
        pltpu.sync_copy(dst_hbm.at[sid], didx)

        plsc.subcore_barrier()
        for j in range(4):
            pltpu.async_copy(ycopy.at[sidx.at[j]], rows[j], gsem[j])

        @pl.loop(0, _CPC // 4 - 1)
        def _(g):
            for j in range(4):
                c = g * 4 + j
                pltpu.make_async_copy(ycopy.at[sidx.at[0]], rows[j], gsem[j]).wait()
                pltpu.async_copy(rows[j], acc.at[didx.at[c]], ssem[j], add=True)
            for j in range(4):
                c4 = g * 4 + 4 + j
                pltpu.make_async_copy(rows[j], acc.at[didx.at[0]], ssem[j]).wait()
                pltpu.async_copy(ycopy.at[sidx.at[c4]], rows[j], gsem[j])

        for j in range(4):
            c = _CPC - 4 + j
            pltpu.make_async_copy(ycopy.at[sidx.at[0]], rows[j], gsem[j]).wait()
            pltpu.async_copy(rows[j], acc.at[didx.at[c]], ssem[j], add=True)
        for j in range(4):
            pltpu.make_async_copy(rows[j], acc.at[didx.at[0]], ssem[j]).wait()

        plsc.subcore_barrier()
        ob = sid * 624
        pltpu.sync_copy(acc.at[pl.ds(ob, 624)], out_hbm.at[cid, pl.ds(ob, 624)])

        @pl.when(sid == 15)
        def _():
            pltpu.sync_copy(acc.at[pl.ds(9984, 16)], out_hbm.at[cid, pl.ds(9984, 16)])

    return k(y2, src3, dst3)


def _sc_deg(dst_both):
    @functools.partial(
        pl.kernel,
        out_type=jax.ShapeDtypeStruct((2, _N, 16), jnp.float32),
        mesh=_mesh(),
        scratch_types=[
            pltpu.VMEM((8, 128), jnp.int32),
            pltpu.VMEM((128, 16), jnp.float32),
            pltpu.VMEM_SHARED((_NPAD, 16), jnp.float32),
        ],
        compiler_params=_SC_PARAMS,
    )
    def k(dst_hbm, out_hbm, didx, rows, acc):
        cid = lax.axis_index("c")
        sid = lax.axis_index("s")
        zero = jnp.zeros((16,), jnp.float32)
        one = jnp.ones((16,), jnp.float32)

        @pl.loop(0, 128)
        def _(r):
            rows[r, pl.ds(0, 16)] = zero

        base = sid * 624
        for t in range(4):
            pltpu.sync_copy(rows, acc.at[pl.ds(base + t * 128, 128)])
        pltpu.sync_copy(rows.at[pl.ds(0, 112)], acc.at[pl.ds(base + 512, 112)])

        @pl.when(sid == 15)
        def _():
            pltpu.sync_copy(rows.at[pl.ds(0, 32)], acc.at[pl.ds(9984, 32)])

        @pl.loop(0, 128)
        def _(r):
            rows[r, pl.ds(0, 16)] = one

        plsc.subcore_barrier()

        @pl.loop(0, 20)
        def _(g):
            pltpu.sync_copy(dst_hbm.at[cid, sid, pl.ds(g * 8, 8)], didx)
            for j in range(8):
                pltpu.sync_copy(rows, acc.at[didx.at[j]], add=True)

        plsc.subcore_barrier()
        ob = sid * 624
        pltpu.sync_copy(acc.at[pl.ds(ob, 624)], out_hbm.at[cid, pl.ds(ob, 624)])

        @pl.when(sid == 15)
        def _():
            pltpu.sync_copy(acc.at[pl.ds(9984, 16)], out_hbm.at[cid, pl.ds(9984, 16)])

    return k(dst_both)


def _lrelu(x):
    return jnp.where(x >= 0, x, 0.01 * x)


def _dinv(deg_ref):
    return lax.rsqrt(jnp.maximum(deg_ref[:, 0:1], 1.0))


def _mlp(x, W1, b1, W2, b2):
    def body(x_ref, w1_ref, b1_ref, w2_ref, b2_ref, o_ref):
        h = jnp.dot(x_ref[...], w1_ref[...], preferred_element_type=jnp.float32, precision=lax.Precision.HIGHEST)
        h = _lrelu(h + b1_ref[...])
        h = jnp.dot(h, w2_ref[...], preferred_element_type=jnp.float32, precision=lax.Precision.HIGHEST)
        o_ref[...] = _lrelu(h + b2_ref[...])

    return pl.pallas_call(
        body,
        grid=(_N // _BLK,),
        in_specs=[
            pl.BlockSpec((_BLK, 128), lambda i: (i, 0)),
            pl.BlockSpec((128, _H), lambda i: (0, 0)),
            pl.BlockSpec((1, _H), lambda i: (0, 0)),
            pl.BlockSpec((_H, _H), lambda i: (0, 0)),
            pl.BlockSpec((1, _H), lambda i: (0, 0)),
        ],
        out_specs=pl.BlockSpec((_BLK, _H), lambda i: (i, 0)),
        out_shape=jax.ShapeDtypeStruct((_N, _H), jnp.float32),
    )(x, W1, b1.reshape(1, _H), W2, b2.reshape(1, _H))


def _prep(h, deg):
    def body(h_ref, deg_ref, y_ref):
        y = h_ref[...] * _dinv(deg_ref)
        y_ref[0] = y[:, :32]
        y_ref[1] = y[:, 32:]

    return pl.pallas_call(
        body,
        grid=(_N // _BLK,),
        in_specs=[
            pl.BlockSpec((_BLK, _H), lambda i: (i, 0)),
            pl.BlockSpec((_BLK, 16), lambda i: (i, 0)),
        ],
        out_specs=pl.BlockSpec((2, _BLK, 32), lambda i: (0, i, 0)),
        out_shape=jax.ShapeDtypeStruct((2, _N, 32), jnp.float32),
    )(h, deg)


def _mid(f, p, deg):
    def body(f_ref, p_ref, deg_ref, f1_ref, y1_ref):
        di = _dinv(deg_ref)
        msg = jnp.concatenate([p_ref[0], p_ref[1]], axis=-1)
        f1 = f_ref[...] - msg * di
        f1_ref[...] = f1
        y1 = f1 * di
        y1_ref[0] = y1[:, :32]
        y1_ref[1] = y1[:, 32:]

    return pl.pallas_call(
        body,
        grid=(_N // _BLK,),
        in_specs=[
            pl.BlockSpec((_BLK, _H), lambda i: (i, 0)),
            pl.BlockSpec((2, _BLK, 32), lambda i: (0, i, 0)),
            pl.BlockSpec((_BLK, 16), lambda i: (i, 0)),
        ],
        out_specs=[
            pl.BlockSpec((_BLK, _H), lambda i: (i, 0)),
            pl.BlockSpec((2, _BLK, 32), lambda i: (0, i, 0)),
        ],
        out_shape=[
            jax.ShapeDtypeStruct((_N, _H), jnp.float32),
            jax.ShapeDtypeStruct((2, _N, 32), jnp.float32),
        ],
    )(f, p, deg)


def _end(h, f1, p, deg, W3s, b3):
    def body(h_ref, f1_ref, p_ref, deg_ref, w3_ref, b3_ref, o_ref):
        msg = jnp.concatenate([p_ref[0], p_ref[1]], axis=-1)
        f2 = f1_ref[...] - msg * _dinv(deg_ref)
        o = jnp.dot(h_ref[...], w3_ref[0], preferred_element_type=jnp.float32, precision=lax.Precision.HIGHEST)
        o += jnp.dot(f1_ref[...], w3_ref[1], preferred_element_type=jnp.float32, precision=lax.Precision.HIGHEST)
        o += jnp.dot(f2, w3_ref[2], preferred_element_type=jnp.float32, precision=lax.Precision.HIGHEST)
        o_ref[...] = o + b3_ref[...]

    return pl.pallas_call(
        body,
        grid=(_N // _BLK,),
        in_specs=[
            pl.BlockSpec((_BLK, _H), lambda i: (i, 0)),
            pl.BlockSpec((_BLK, _H), lambda i: (i, 0)),
            pl.BlockSpec((2, _BLK, 32), lambda i: (0, i, 0)),
            pl.BlockSpec((_BLK, 16), lambda i: (i, 0)),
            pl.BlockSpec((3, _H, _H), lambda i: (0, 0, 0)),
            pl.BlockSpec((1, _H), lambda i: (0, 0)),
        ],
        out_specs=pl.BlockSpec((_BLK, _H), lambda i: (i, 0)),
        out_shape=jax.ShapeDtypeStruct((_N, _H), jnp.float32),
    )(h, f1, p, deg, W3s, b3)


def _final(ha, hb, W4p, b4p):
    def body(a_ref, b_ref, w_ref, bias_ref, o_ref):
        s = _lrelu(a_ref[...] + b_ref[...])
        o_ref[...] = (
            jnp.dot(s, w_ref[...], preferred_element_type=jnp.float32, precision=lax.Precision.HIGHEST)
            + bias_ref[...]
        )

    return pl.pallas_call(
        body,
        grid=(_N // _BLK,),
        in_specs=[
            pl.BlockSpec((_BLK, _H), lambda i: (i, 0)),
            pl.BlockSpec((_BLK, _H), lambda i: (i, 0)),
            pl.BlockSpec((_H, 128), lambda i: (0, 0)),
            pl.BlockSpec((1, 128), lambda i: (0, 0)),
        ],
        out_specs=pl.BlockSpec((_BLK, 128), lambda i: (i, 0)),
        out_shape=jax.ShapeDtypeStruct((_N, 128), jnp.float32),
    )(ha, hb, W4p, b4p)


def kernel(in_feat, edge_index_r0, edge_index_r1, W1, b1, W2, b2, W3, b3, W4, b4):
    pad = _EPAD - _E

    def prep_edges(ei):
        src = jnp.concatenate([ei[0], jnp.zeros((pad,), jnp.int32)])
        dst = jnp.concatenate([ei[1], jnp.full((pad,), _N, jnp.int32)])
        return src.reshape(16, _CPC, 128), dst.reshape(16, _CPC, 128)

    src0, dst0 = prep_edges(edge_index_r0)
    src1, dst1 = prep_edges(edge_index_r1)

    deg_both = _sc_deg(jnp.stack([dst0, dst1]))
    h = _mlp(in_feat, W1, b1, W2, b2)

    W3s = jnp.stack([
        3.0 * W3[:_H],
        -3.0 * W3[:_H] + 3.0 * W3[_H:2 * _H],
        0.75 * W3[:_H] - 1.5 * W3[_H:2 * _H] + 0.75 * W3[2 * _H:],
    ])
    b3r = b3.reshape(1, _H)

    h_all = []
    for r, (src3, dst3) in enumerate(((src0, dst0), (src1, dst1))):
        deg = deg_both[r]
        y0 = _prep(h, deg)
        p1 = _sc_segsum(y0, src3, dst3)
        f1, y1 = _mid(h, p1, deg)
        p2 = _sc_segsum(y1, src3, dst3)
        h = _end(h, f1, p2, deg, W3s, b3r)
        h_all.append(h)

    W4p = jnp.zeros((_H, 128), jnp.float32).at[:, :2].set(W4)
    b4p = jnp.zeros((1, 128), jnp.float32).at[0, :2].set(b4)
    out = _final(h_all[0], h_all[1], W4p, b4p)
    return out[:, :2]


# speedup vs baseline: 9.0720x; 1.0366x over previous
import functools

import jax
import jax.numpy as jnp
from jax import lax
from jax.experimental import pallas as pl
from jax.experimental.pallas import tpu as pltpu
from jax.experimental.pallas import tpu_sc as plsc

_N = 10000
_E = 320000
_H = 64
_NPAD = 10016
_EPAD = 327680
_NTILES = 32
_CPC = 160
_BLK = 1000


def _mesh():
    return plsc.VectorSubcoreMesh(core_axis_name="c", subcore_axis_name="s")


_SC_PARAMS = pltpu.CompilerParams(use_tc_tiling_on_sc=False)


def _sc_segsum(y2, src3, dst3):
    @functools.partial(
        pl.kernel,
        out_type=jax.ShapeDtypeStruct((2, _N, 32), jnp.float32),
        mesh=_mesh(),
        scratch_types=[
            pltpu.VMEM((160, 128), jnp.int32),
            pltpu.VMEM((160, 128), jnp.int32),
            pltpu.VMEM((128, 32), jnp.float32),
            pltpu.VMEM((128, 32), jnp.float32),
            pltpu.VMEM((128, 32), jnp.float32),
            pltpu.VMEM((128, 32), jnp.float32),
            pltpu.VMEM((128, 32), jnp.float32),
            pltpu.VMEM((128, 32), jnp.float32),
            pltpu.VMEM((128, 32), jnp.float32),
            pltpu.VMEM((128, 32), jnp.float32),
            pltpu.VMEM_SHARED((_NPAD, 32), jnp.float32),
            pltpu.VMEM_SHARED((_N, 32), jnp.float32),
        ] + [pltpu.SemaphoreType.DMA] * 16,
        compiler_params=_SC_PARAMS,
    )
    def k(y_hbm, src_hbm, dst_hbm, out_hbm, sidx, didx,
          r0, r1, r2, r3, r4, r5, r6, r7, acc, ycopy, *sems):
        cid = lax.axis_index("c")
        sid = lax.axis_index("s")
        rows = (r0, r1, r2, r3, r4, r5, r6, r7)
        gsem = sems[:8]
        ssem = sems[8:]
        zero = jnp.zeros((16,), jnp.float32)

        @pl.loop(0, 128)
        def _(r):
            for c2 in range(2):
                r0[r, pl.ds(c2 * 16, 16)] = zero

        base = sid * 624
        for t in range(4):
            pltpu.sync_copy(r0, acc.at[pl.ds(base + t * 128, 128)])
        pltpu.sync_copy(r0.at[pl.ds(0, 112)], acc.at[pl.ds(base + 512, 112)])

        @pl.when(sid == 15)
        def _():
            pltpu.sync_copy(r0.at[pl.ds(0, 32)], acc.at[pl.ds(9984, 32)])

        pltpu.sync_copy(y_hbm.at[cid, pl.ds(base, 624)], ycopy.at[pl.ds(base, 624)])

        @pl.when(sid == 15)
        def _():
            pltpu.sync_copy(y_hbm.at[cid, pl.ds(9984, 16)], ycopy.at[pl.ds(9984, 16)])

        pltpu.sync_copy(src_hbm.at[sid], sidx)
        pltpu.sync_copy(dst_hbm.at[sid], didx)

        plsc.subcore_barrier()
        for j in range(8):
            pltpu.async_copy(ycopy.at[sidx.at[j]], rows[j], gsem[j])

        @pl.loop(0, _CPC // 8 - 1)
        def _(g):
            for j in range(8):
                c = g * 8 + j
                pltpu.make_async_copy(ycopy.at[sidx.at[0]], rows[j], gsem[j]).wait()
                pltpu.async_copy(rows[j], acc.at[didx.at[c]], ssem[j], add=True)
            for j in range(8):
                c8 = g * 8 + 8 + j
                pltpu.make_async_copy(rows[j], acc.at[didx.at[0]], ssem[j]).wait()
                pltpu.async_copy(ycopy.at[sidx.at[c8]], rows[j], gsem[j])

        for j in range(8):
            c = _CPC - 8 + j
            pltpu.make_async_copy(ycopy.at[sidx.at[0]], rows[j], gsem[j]).wait()
            pltpu.async_copy(rows[j], acc.at[didx.at[c]], ssem[j], add=True)
        for j in range(8):
            pltpu.make_async_copy(rows[j], acc.at[didx.at[0]], ssem[j]).wait()

        plsc.subcore_barrier()
        ob = sid * 624
        pltpu.sync_copy(acc.at[pl.ds(ob, 624)], out_hbm.at[cid, pl.ds(ob, 624)])

        @pl.when(sid == 15)
        def _():
            pltpu.sync_copy(acc.at[pl.ds(9984, 16)], out_hbm.at[cid, pl.ds(9984, 16)])

    return k(y2, src3, dst3)


def _sc_deg(dst_both):
    @functools.partial(
        pl.kernel,
        out_type=jax.ShapeDtypeStruct((2, _N, 16), jnp.float32),
        mesh=_mesh(),
        scratch_types=[
            pltpu.VMEM((8, 128), jnp.int32),
            pltpu.VMEM((128, 16), jnp.float32),
            pltpu.VMEM_SHARED((_NPAD, 16), jnp.float32),
        ],
        compiler_params=_SC_PARAMS,
    )
    def k(dst_hbm, out_hbm, didx, rows, acc):
        cid = lax.axis_index("c")
        sid = lax.axis_index("s")
        zero = jnp.zeros((16,), jnp.float32)
        one = jnp.ones((16,), jnp.float32)

        @pl.loop(0, 128)
        def _(r):
            rows[r, pl.ds(0, 16)] = zero

        base = sid * 624
        for t in range(4):
            pltpu.sync_copy(rows, acc.at[pl.ds(base + t * 128, 128)])
        pltpu.sync_copy(rows.at[pl.ds(0, 112)], acc.at[pl.ds(base + 512, 112)])

        @pl.when(sid == 15)
        def _():
            pltpu.sync_copy(rows.at[pl.ds(0, 32)], acc.at[pl.ds(9984, 32)])

        @pl.loop(0, 128)
        def _(r):
            rows[r, pl.ds(0, 16)] = one

        plsc.subcore_barrier()

        @pl.loop(0, 20)
        def _(g):
            pltpu.sync_copy(dst_hbm.at[cid, sid, pl.ds(g * 8, 8)], didx)
            for j in range(8):
                pltpu.sync_copy(rows, acc.at[didx.at[j]], add=True)

        plsc.subcore_barrier()
        ob = sid * 624
        pltpu.sync_copy(acc.at[pl.ds(ob, 624)], out_hbm.at[cid, pl.ds(ob, 624)])

        @pl.when(sid == 15)
        def _():
            pltpu.sync_copy(acc.at[pl.ds(9984, 16)], out_hbm.at[cid, pl.ds(9984, 16)])

    return k(dst_both)


def _lrelu(x):
    return jnp.where(x >= 0, x, 0.01 * x)


def _dinv(deg_ref):
    return lax.rsqrt(jnp.maximum(deg_ref[:, 0:1], 1.0))


def _mlp(x, W1, b1, W2, b2):
    def body(x_ref, w1_ref, b1_ref, w2_ref, b2_ref, o_ref):
        h = jnp.dot(x_ref[...], w1_ref[...], preferred_element_type=jnp.float32, precision=lax.Precision.HIGHEST)
        h = _lrelu(h + b1_ref[...])
        h = jnp.dot(h, w2_ref[...], preferred_element_type=jnp.float32, precision=lax.Precision.HIGHEST)
        o_ref[...] = _lrelu(h + b2_ref[...])

    return pl.pallas_call(
        body,
        grid=(_N // _BLK,),
        in_specs=[
            pl.BlockSpec((_BLK, 128), lambda i: (i, 0)),
            pl.BlockSpec((128, _H), lambda i: (0, 0)),
            pl.BlockSpec((1, _H), lambda i: (0, 0)),
            pl.BlockSpec((_H, _H), lambda i: (0, 0)),
            pl.BlockSpec((1, _H), lambda i: (0, 0)),
        ],
        out_specs=pl.BlockSpec((_BLK, _H), lambda i: (i, 0)),
        out_shape=jax.ShapeDtypeStruct((_N, _H), jnp.float32),
    )(x, W1, b1.reshape(1, _H), W2, b2.reshape(1, _H))


def _prep(h, deg):
    def body(h_ref, deg_ref, y_ref):
        y = h_ref[...] * _dinv(deg_ref)
        y_ref[0] = y[:, :32]
        y_ref[1] = y[:, 32:]

    return pl.pallas_call(
        body,
        grid=(_N // _BLK,),
        in_specs=[
            pl.BlockSpec((_BLK, _H), lambda i: (i, 0)),
            pl.BlockSpec((_BLK, 16), lambda i: (i, 0)),
        ],
        out_specs=pl.BlockSpec((2, _BLK, 32), lambda i: (0, i, 0)),
        out_shape=jax.ShapeDtypeStruct((2, _N, 32), jnp.float32),
    )(h, deg)


def _mid(f, p, deg):
    def body(f_ref, p_ref, deg_ref, f1_ref, y1_ref):
        di = _dinv(deg_ref)
        msg = jnp.concatenate([p_ref[0], p_ref[1]], axis=-1)
        f1 = f_ref[...] - msg * di
        f1_ref[...] = f1
        y1 = f1 * di
        y1_ref[0] = y1[:, :32]
        y1_ref[1] = y1[:, 32:]

    return pl.pallas_call(
        body,
        grid=(_N // _BLK,),
        in_specs=[
            pl.BlockSpec((_BLK, _H), lambda i: (i, 0)),
            pl.BlockSpec((2, _BLK, 32), lambda i: (0, i, 0)),
            pl.BlockSpec((_BLK, 16), lambda i: (i, 0)),
        ],
        out_specs=[
            pl.BlockSpec((_BLK, _H), lambda i: (i, 0)),
            pl.BlockSpec((2, _BLK, 32), lambda i: (0, i, 0)),
        ],
        out_shape=[
            jax.ShapeDtypeStruct((_N, _H), jnp.float32),
            jax.ShapeDtypeStruct((2, _N, 32), jnp.float32),
        ],
    )(f, p, deg)


def _end(h, f1, p, deg, W3s, b3):
    def body(h_ref, f1_ref, p_ref, deg_ref, w3_ref, b3_ref, o_ref):
        msg = jnp.concatenate([p_ref[0], p_ref[1]], axis=-1)
        f2 = f1_ref[...] - msg * _dinv(deg_ref)
        o = jnp.dot(h_ref[...], w3_ref[0], preferred_element_type=jnp.float32, precision=lax.Precision.HIGHEST)
        o += jnp.dot(f1_ref[...], w3_ref[1], preferred_element_type=jnp.float32, precision=lax.Precision.HIGHEST)
        o += jnp.dot(f2, w3_ref[2], preferred_element_type=jnp.float32, precision=lax.Precision.HIGHEST)
        o_ref[...] = o + b3_ref[...]

    return pl.pallas_call(
        body,
        grid=(_N // _BLK,),
        in_specs=[
            pl.BlockSpec((_BLK, _H), lambda i: (i, 0)),
            pl.BlockSpec((_BLK, _H), lambda i: (i, 0)),
            pl.BlockSpec((2, _BLK, 32), lambda i: (0, i, 0)),
            pl.BlockSpec((_BLK, 16), lambda i: (i, 0)),
            pl.BlockSpec((3, _H, _H), lambda i: (0, 0, 0)),
            pl.BlockSpec((1, _H), lambda i: (0, 0)),
        ],
        out_specs=pl.BlockSpec((_BLK, _H), lambda i: (i, 0)),
        out_shape=jax.ShapeDtypeStruct((_N, _H), jnp.float32),
    )(h, f1, p, deg, W3s, b3)


def _final(ha, hb, W4p, b4p):
    def body(a_ref, b_ref, w_ref, bias_ref, o_ref):
        s = _lrelu(a_ref[...] + b_ref[...])
        o_ref[...] = (
            jnp.dot(s, w_ref[...], preferred_element_type=jnp.float32, precision=lax.Precision.HIGHEST)
            + bias_ref[...]
        )

    return pl.pallas_call(
        body,
        grid=(_N // _BLK,),
        in_specs=[
            pl.BlockSpec((_BLK, _H), lambda i: (i, 0)),
            pl.BlockSpec((_BLK, _H), lambda i: (i, 0)),
            pl.BlockSpec((_H, 128), lambda i: (0, 0)),
            pl.BlockSpec((1, 128), lambda i: (0, 0)),
        ],
        out_specs=pl.BlockSpec((_BLK, 128), lambda i: (i, 0)),
        out_shape=jax.ShapeDtypeStruct((_N, 128), jnp.float32),
    )(ha, hb, W4p, b4p)


def kernel(in_feat, edge_index_r0, edge_index_r1, W1, b1, W2, b2, W3, b3, W4, b4):
    pad = _EPAD - _E

    def prep_edges(ei):
        src = jnp.concatenate([ei[0], jnp.zeros((pad,), jnp.int32)])
        dst = jnp.concatenate([ei[1], jnp.full((pad,), _N, jnp.int32)])
        return src.reshape(16, _CPC, 128), dst.reshape(16, _CPC, 128)

    src0, dst0 = prep_edges(edge_index_r0)
    src1, dst1 = prep_edges(edge_index_r1)

    deg_both = _sc_deg(jnp.stack([dst0, dst1]))
    h = _mlp(in_feat, W1, b1, W2, b2)

    W3s = jnp.stack([
        3.0 * W3[:_H],
        -3.0 * W3[:_H] + 3.0 * W3[_H:2 * _H],
        0.75 * W3[:_H] - 1.5 * W3[_H:2 * _H] + 0.75 * W3[2 * _H:],
    ])
    b3r = b3.reshape(1, _H)

    h_all = []
    for r, (src3, dst3) in enumerate(((src0, dst0), (src1, dst1))):
        deg = deg_both[r]
        y0 = _prep(h, deg)
        p1 = _sc_segsum(y0, src3, dst3)
        f1, y1 = _mid(h, p1, deg)
        p2 = _sc_segsum(y1, src3, dst3)
        h = _end(h, f1, p2, deg, W3s, b3r)
        h_all.append(h)

    W4p = jnp.zeros((_H, 128), jnp.float32).at[:, :2].set(W4)
    b4p = jnp.zeros((1, 128), jnp.float32).at[0, :2].set(b4)
    out = _final(h_all[0], h_all[1], W4p, b4p)
    return out[:, :2]
